# Initial kernel scaffold; baseline (speedup 1.0000x reference)
import jax
import jax.numpy as jnp
from jax.experimental import pallas as pl


def kernel(x, edge_index, batch, Wg1, al1, ar1, Wg2, al2, ar2, W1, b1, W2, b2):
    raise NotImplementedError("write your pallas kernel here")


# speedup vs baseline: 12.7764x; 12.7764x over previous
import functools

import jax
import jax.numpy as jnp
from jax import lax
from jax.experimental import pallas as pl
from jax.experimental.pallas import tpu as pltpu
from jax.experimental.pallas import tpu_sc as plsc

N = 10000
E = 320000
H = 128
HH = H // 2
HQ = H // 4
G = 64
C = 2

NCORES = 2
NTILES = 16
ET = E // NTILES
NR = N // 16
NRP = 640
KC = 80
RB = 624

_SELU_L = 1.0507009873554805
_SELU_A = 1.6732632423543772


def _selu(v):
    return _SELU_L * jnp.where(v > 0, v, _SELU_A * (jnp.exp(v) - 1.0))


def _entry1_body(x_ref, w_ref, aw_ref, h_ref, aux_ref):
    h = jnp.dot(x_ref[...], w_ref[...], preferred_element_type=jnp.float32)
    h_ref[...] = h
    aux_ref[...] = jnp.dot(h, aw_ref[...], preferred_element_type=jnp.float32)


def _entry2_body(p0_ref, p1_ref, p2_ref, p3_ref, w0_ref, w1_ref, w2_ref,
                 w3_ref, aw_ref, h_ref, aux_ref):
    h = (jnp.dot(_selu(p0_ref[...]), w0_ref[...],
                 preferred_element_type=jnp.float32)
         + jnp.dot(_selu(p1_ref[...]), w1_ref[...],
                   preferred_element_type=jnp.float32)
         + jnp.dot(_selu(p2_ref[...]), w2_ref[...],
                   preferred_element_type=jnp.float32)
         + jnp.dot(_selu(p3_ref[...]), w3_ref[...],
                   preferred_element_type=jnp.float32))
    h_ref[...] = h
    aux_ref[...] = jnp.dot(h, aw_ref[...], preferred_element_type=jnp.float32)


def _tc_entry1(x, w, aw):
    return pl.pallas_call(
        _entry1_body,
        grid=(5,),
        in_specs=[
            pl.BlockSpec((2000, H), lambda i: (i, 0)),
            pl.BlockSpec((H, H), lambda i: (0, 0)),
            pl.BlockSpec((H, 8), lambda i: (0, 0)),
        ],
        out_specs=[
            pl.BlockSpec((2000, H), lambda i: (i, 0)),
            pl.BlockSpec((2000, 8), lambda i: (i, 0)),
        ],
        out_shape=[
            jax.ShapeDtypeStruct((N, H), jnp.float32),
            jax.ShapeDtypeStruct((N, 8), jnp.float32),
        ],
    )(x, w, aw)


def _tc_entry2(p, w, aw):
    return pl.pallas_call(
        _entry2_body,
        grid=(5,),
        in_specs=[
            pl.BlockSpec((2000, HQ), lambda i: (i, 0)),
            pl.BlockSpec((2000, HQ), lambda i: (i + 5, 0)),
            pl.BlockSpec((2000, HQ), lambda i: (i + 10, 0)),
            pl.BlockSpec((2000, HQ), lambda i: (i + 15, 0)),
            pl.BlockSpec((HQ, H), lambda i: (0, 0)),
            pl.BlockSpec((HQ, H), lambda i: (1, 0)),
            pl.BlockSpec((HQ, H), lambda i: (2, 0)),
            pl.BlockSpec((HQ, H), lambda i: (3, 0)),
            pl.BlockSpec((H, 8), lambda i: (0, 0)),
        ],
        out_specs=[
            pl.BlockSpec((2000, H), lambda i: (i, 0)),
            pl.BlockSpec((2000, 8), lambda i: (i, 0)),
        ],
        out_shape=[
            jax.ShapeDtypeStruct((N, H), jnp.float32),
            jax.ShapeDtypeStruct((N, 8), jnp.float32),
        ],
    )(p, p, p, p, w, w, w, w, aw)


def _final_body(p0_ref, p1_ref, p2_ref, p3_ref, b_ref, w1_ref, b1_ref,
                w2_ref, b2_ref, o_ref):
    a = jnp.concatenate([_selu(p0_ref[...]), _selu(p1_ref[...]),
                         _selu(p2_ref[...]), _selu(p3_ref[...])], axis=1)
    bt = b_ref[...]
    gid = lax.broadcasted_iota(jnp.int32, (G, N), 0).astype(jnp.float32)
    P = jnp.where(gid == bt, 1.0, 0.0).astype(jnp.float32)
    cnt = jnp.sum(P, axis=1, keepdims=True)
    pooled = jnp.dot(P, a, preferred_element_type=jnp.float32)
    pooled = pooled / jnp.maximum(cnt, 1.0)
    hm = _selu(jnp.dot(pooled, w1_ref[...],
                       preferred_element_type=jnp.float32) + b1_ref[...])
    logits = jnp.dot(hm, w2_ref[...],
                     preferred_element_type=jnp.float32) + b2_ref[...]
    mx = jnp.max(logits, axis=1, keepdims=True)
    z = logits - mx
    o_ref[...] = z - jnp.log(jnp.sum(jnp.exp(z), axis=1, keepdims=True))


def _tc_final(p, batch2d, w1, b1, w2, b2):
    return pl.pallas_call(
        _final_body,
        grid=(1,),
        in_specs=[
            pl.BlockSpec((N, HQ), lambda i: (0, 0)),
            pl.BlockSpec((N, HQ), lambda i: (1, 0)),
            pl.BlockSpec((N, HQ), lambda i: (2, 0)),
            pl.BlockSpec((N, HQ), lambda i: (3, 0)),
            pl.BlockSpec((1, N), lambda i: (0, 0)),
            pl.BlockSpec((H, H), lambda i: (0, 0)),
            pl.BlockSpec((1, H), lambda i: (0, 0)),
            pl.BlockSpec((H, C), lambda i: (0, 0)),
            pl.BlockSpec((1, C), lambda i: (0, 0)),
        ],
        out_specs=pl.BlockSpec((G, C), lambda i: (0, 0)),
        out_shape=jax.ShapeDtypeStruct((G, C), jnp.float32),
    )(p, p, p, p, batch2d, w1, b1, w2, b2)


_GAT_KERNEL_CACHE = []


def _build_gat_kernel():
    mesh = plsc.VectorSubcoreMesh(core_axis_name="c", subcore_axis_name="s")

    @functools.partial(
        pl.kernel,
        out_type=jax.ShapeDtypeStruct((4 * N, HQ), jnp.float32),
        mesh=mesh,
        compiler_params=pltpu.CompilerParams(
            needs_layout_passes=False, use_tc_tiling_on_sc=False),
        scratch_types=[
            pltpu.VMEM((NR, 16), jnp.float32),
            pltpu.VMEM((NR, 16), jnp.float32),
            pltpu.VMEM((NRP, 16), jnp.float32),
            pltpu.VMEM((ET,), jnp.int32),
            pltpu.VMEM((ET,), jnp.int32),
            pltpu.VMEM((ET // 16, 16), jnp.float32),
            pltpu.VMEM((KC, HQ), jnp.float32),
            pltpu.VMEM((KC,), jnp.int32),
            pltpu.VMEM((KC,), jnp.int32),
            pltpu.VMEM((NRP,), jnp.int32),
            pltpu.VMEM((16, 16), jnp.float32),
            pltpu.VMEM_SHARED((NRP, 16), jnp.float32),
            pltpu.VMEM_SHARED((N, HQ), jnp.float32),
            pltpu.SemaphoreType.DMA,
        ],
    )
    def k(hq0_hbm, hq1_hbm, hq2_hbm, hq3_hbm, src_hbm, dst_hbm,
          el_hbm, er_hbm, out_hbm,
          el_v, er_v, den_v, src_v, dst_v, ee_v, rows_v, sidx_v, didx_v,
          irow_v, scr_v, dens_s, outs_s, sem):
        c = lax.axis_index("c")
        t = lax.axis_index("s")
        i16 = lax.iota(jnp.int32, 16)
        zf16 = jnp.zeros((16,), jnp.float32)

        pltpu.sync_copy(el_hbm, el_v)
        pltpu.sync_copy(er_hbm, er_v)
        eb = t * ET
        pltpu.sync_copy(src_hbm.at[pl.ds(eb, ET)], src_v)
        pltpu.sync_copy(dst_hbm.at[pl.ds(eb, ET)], dst_v)

        def zden(r, _):
            den_v[r, :] = zf16
            return 0
        lax.fori_loop(0, NRP, zden, 0)

        def zrows(r, _):
            for j in range(HQ // 16):
                rows_v[r, pl.ds(16 * j, 16)] = zf16
            return 0
        lax.fori_loop(0, KC, zrows, 0)

        for g in range(NRP // 16):
            irow_v[pl.ds(16 * g, 16)] = i16 + 16 * g

        @pl.when(t == 0)
        def _():
            pltpu.sync_copy(den_v, dens_s)

        def mrow(r, carry):
            ml, mr = carry
            return (jnp.maximum(ml, el_v[r, :]), jnp.maximum(mr, er_v[r, :]))
        accl, accr = lax.fori_loop(
            0, NR, mrow,
            (jnp.full((16,), -3e38, jnp.float32),
             jnp.full((16,), -3e38, jnp.float32)))
        z16 = jnp.zeros((16,), jnp.int32)
        scr_v[0, :] = accl
        scr_v[1, :] = accr
        ml = jnp.full((16,), -3e38, jnp.float32)
        mr = jnp.full((16,), -3e38, jnp.float32)
        for j in range(16):
            jj = jnp.full((16,), j, jnp.int32)
            ml = jnp.maximum(ml, plsc.load_gather(scr_v, [z16, jj]))
            mr = jnp.maximum(mr, plsc.load_gather(scr_v, [z16 + 1, jj]))
        msum = ml + mr
        m = jnp.maximum(msum, 0.2 * msum) - 30.0

        plsc.subcore_barrier()

        def pa(g, _):
            s16 = src_v[pl.ds(g * 16, 16)]
            d16 = dst_v[pl.ds(g * 16, 16)]
            srow = lax.shift_right_logical(s16, 4)
            scol = lax.bitwise_and(s16, 15)
            drow = lax.shift_right_logical(d16, 4)
            dcol = lax.bitwise_and(d16, 15)
            elv = plsc.load_gather(el_v, [srow, scol])
            erv = plsc.load_gather(er_v, [drow, dcol])
            xv = elv + erv
            ev = jnp.where(xv > 0, xv, 0.2 * xv)
            ee = jnp.exp(ev - m)
            ee_v[g, :] = ee
            plsc.addupdate_scatter(den_v, [drow, dcol], ee)
            return 0
        lax.fori_loop(0, ET // 16, pa, 0)

        pltpu.sync_copy(den_v, dens_s.at[irow_v], add=True)
        plsc.subcore_barrier()
        pltpu.sync_copy(dens_s, den_v)

        def pb(g, _):
            ee = ee_v[g, :]
            d16 = dst_v[pl.ds(g * 16, 16)]
            drow = lax.shift_right_logical(d16, 4)
            dcol = lax.bitwise_and(d16, 15)
            dv = plsc.load_gather(den_v, [drow, dcol])
            ee_v[g, :] = ee / (dv + 1e-9)
            return 0
        lax.fori_loop(0, ET // 16, pb, 0)

        ob = t * RB
        for half in range(2):
            for q in range(RB // KC):
                pltpu.sync_copy(rows_v, outs_s.at[pl.ds(ob + q * KC, KC)])
            rem = RB - (RB // KC) * KC
            if rem:
                pltpu.sync_copy(rows_v.at[pl.ds(0, rem)],
                                outs_s.at[pl.ds(ob + (RB // KC) * KC, rem)])

            @pl.when(t == NTILES - 1)
            def _():
                pltpu.sync_copy(
                    rows_v.at[pl.ds(0, N - NTILES * RB)],
                    outs_s.at[pl.ds(NTILES * RB, N - NTILES * RB)])

            plsc.subcore_barrier()

            def pc(kk, _):
                off = kk * KC
                for u in range(KC // 16):
                    sidx_v[pl.ds(16 * u, 16)] = src_v[pl.ds(off + 16 * u, 16)]
                    didx_v[pl.ds(16 * u, 16)] = dst_v[pl.ds(off + 16 * u, 16)]

                if half == 0:
                    @pl.when(c == 0)
                    def _():
                        pltpu.async_copy(
                            hq0_hbm.at[sidx_v], rows_v, sem).wait()

                    @pl.when(c == 1)
                    def _():
                        pltpu.async_copy(
                            hq2_hbm.at[sidx_v], rows_v, sem).wait()
                else:
                    @pl.when(c == 0)
                    def _():
                        pltpu.async_copy(
                            hq1_hbm.at[sidx_v], rows_v, sem).wait()

                    @pl.when(c == 1)
                    def _():
                        pltpu.async_copy(
                            hq3_hbm.at[sidx_v], rows_v, sem).wait()

                def scale(i, _):
                    ea = off + i
                    av = plsc.load_gather(
                        ee_v, [jnp.full((16,), lax.shift_right_logical(ea, 4),
                                        jnp.int32),
                               jnp.full((16,), lax.bitwise_and(ea, 15),
                                        jnp.int32)])
                    for j in range(HQ // 16):
                        blk = rows_v[i, pl.ds(16 * j, 16)]
                        rows_v[i, pl.ds(16 * j, 16)] = blk * av
                    return 0
                lax.fori_loop(0, KC, scale, 0)

                pltpu.sync_copy(rows_v, outs_s.at[didx_v], add=True)
                return 0
            lax.fori_loop(0, ET // KC, pc, 0)
            plsc.subcore_barrier()

            qb = (2 * c + half) * N
            pltpu.sync_copy(outs_s.at[pl.ds(ob, RB)],
                            out_hbm.at[pl.ds(qb + ob, RB)])

            @pl.when(t == NTILES - 1)
            def _():
                pltpu.sync_copy(
                    outs_s.at[pl.ds(NTILES * RB, N - NTILES * RB)],
                    out_hbm.at[pl.ds(qb + NTILES * RB, N - NTILES * RB)])

            plsc.subcore_barrier()

    return k


def _gat_sc(hq0, hq1, hq2, hq3, src, dst, el2d, er2d):
    if not _GAT_KERNEL_CACHE:
        _GAT_KERNEL_CACHE.append(_build_gat_kernel())
    return _GAT_KERNEL_CACHE[0](hq0, hq1, hq2, hq3, src, dst, el2d, er2d)


def kernel(x, edge_index, batch, Wg1, al1, ar1, Wg2, al2, ar2, W1, b1, W2, b2):
    src = edge_index[0]
    dst = edge_index[1]
    aw1 = jnp.zeros((H, 8), jnp.float32).at[:, 0].set(al1).at[:, 1].set(ar1)
    aw2 = jnp.zeros((H, 8), jnp.float32).at[:, 0].set(al2).at[:, 1].set(ar2)

    h1, aux1 = _tc_entry1(x, Wg1, aw1)
    el1 = aux1[:, 0].reshape(NR, 16)
    er1 = aux1[:, 1].reshape(NR, 16)
    p1 = _gat_sc(h1[:, :HQ], h1[:, HQ:2 * HQ], h1[:, 2 * HQ:3 * HQ],
                 h1[:, 3 * HQ:], src, dst, el1, er1)

    h2, aux2 = _tc_entry2(p1, Wg2, aw2)
    el2 = aux2[:, 0].reshape(NR, 16)
    er2 = aux2[:, 1].reshape(NR, 16)
    p2 = _gat_sc(h2[:, :HQ], h2[:, HQ:2 * HQ], h2[:, 2 * HQ:3 * HQ],
                 h2[:, 3 * HQ:], src, dst, el2, er2)

    batchf = batch.astype(jnp.float32).reshape(1, N)
    return _tc_final(p2, batchf, W1,
                     b1.reshape(1, H), W2, b2.reshape(1, C))


# speedup vs baseline: 21.1755x; 1.6574x over previous
import functools

import jax
import jax.numpy as jnp
from jax import lax
from jax.experimental import pallas as pl
from jax.experimental.pallas import tpu as pltpu
from jax.experimental.pallas import tpu_sc as plsc

N = 10000
E = 320000
H = 128
HH = H // 2
HQ = H // 4
G = 64
C = 2

NCORES = 2
NTILES = 16
ET = E // NTILES
NR = N // 16
NRP = 640
KC = 80
RB = 624

_SELU_L = 1.0507009873554805
_SELU_A = 1.6732632423543772


def _selu(v):
    return _SELU_L * jnp.where(v > 0, v, _SELU_A * (jnp.exp(v) - 1.0))


def _entry1_body(x_ref, w_ref, aw_ref, h_ref, aux_ref):
    h = jnp.dot(x_ref[...], w_ref[...], preferred_element_type=jnp.float32)
    h_ref[...] = h
    aux_ref[...] = jnp.dot(h, aw_ref[...], preferred_element_type=jnp.float32)


def _entry2_body(p0_ref, p1_ref, p2_ref, p3_ref, w0_ref, w1_ref, w2_ref,
                 w3_ref, aw_ref, h_ref, aux_ref):
    h = (jnp.dot(_selu(p0_ref[...]), w0_ref[...],
                 preferred_element_type=jnp.float32)
         + jnp.dot(_selu(p1_ref[...]), w1_ref[...],
                   preferred_element_type=jnp.float32)
         + jnp.dot(_selu(p2_ref[...]), w2_ref[...],
                   preferred_element_type=jnp.float32)
         + jnp.dot(_selu(p3_ref[...]), w3_ref[...],
                   preferred_element_type=jnp.float32))
    h_ref[...] = h
    aux_ref[...] = jnp.dot(h, aw_ref[...], preferred_element_type=jnp.float32)


def _tc_entry1(x, w, aw):
    return pl.pallas_call(
        _entry1_body,
        grid=(5,),
        in_specs=[
            pl.BlockSpec((2000, H), lambda i: (i, 0)),
            pl.BlockSpec((H, H), lambda i: (0, 0)),
            pl.BlockSpec((H, 8), lambda i: (0, 0)),
        ],
        out_specs=[
            pl.BlockSpec((2000, H), lambda i: (i, 0)),
            pl.BlockSpec((2000, 8), lambda i: (i, 0)),
        ],
        out_shape=[
            jax.ShapeDtypeStruct((N, H), jnp.float32),
            jax.ShapeDtypeStruct((N, 8), jnp.float32),
        ],
    )(x, w, aw)


def _tc_entry2(p, w, aw):
    return pl.pallas_call(
        _entry2_body,
        grid=(5,),
        in_specs=[
            pl.BlockSpec((2000, HQ), lambda i: (i, 0)),
            pl.BlockSpec((2000, HQ), lambda i: (i + 5, 0)),
            pl.BlockSpec((2000, HQ), lambda i: (i + 10, 0)),
            pl.BlockSpec((2000, HQ), lambda i: (i + 15, 0)),
            pl.BlockSpec((HQ, H), lambda i: (0, 0)),
            pl.BlockSpec((HQ, H), lambda i: (1, 0)),
            pl.BlockSpec((HQ, H), lambda i: (2, 0)),
            pl.BlockSpec((HQ, H), lambda i: (3, 0)),
            pl.BlockSpec((H, 8), lambda i: (0, 0)),
        ],
        out_specs=[
            pl.BlockSpec((2000, H), lambda i: (i, 0)),
            pl.BlockSpec((2000, 8), lambda i: (i, 0)),
        ],
        out_shape=[
            jax.ShapeDtypeStruct((N, H), jnp.float32),
            jax.ShapeDtypeStruct((N, 8), jnp.float32),
        ],
    )(p, p, p, p, w, w, w, w, aw)


def _final_body(p0_ref, p1_ref, p2_ref, p3_ref, b_ref, w1_ref, b1_ref,
                w2_ref, b2_ref, o_ref):
    a = jnp.concatenate([_selu(p0_ref[...]), _selu(p1_ref[...]),
                         _selu(p2_ref[...]), _selu(p3_ref[...])], axis=1)
    bt = b_ref[...]
    gid = lax.broadcasted_iota(jnp.int32, (G, N), 0).astype(jnp.float32)
    P = jnp.where(gid == bt, 1.0, 0.0).astype(jnp.float32)
    cnt = jnp.sum(P, axis=1, keepdims=True)
    pooled = jnp.dot(P, a, preferred_element_type=jnp.float32)
    pooled = pooled / jnp.maximum(cnt, 1.0)
    hm = _selu(jnp.dot(pooled, w1_ref[...],
                       preferred_element_type=jnp.float32) + b1_ref[...])
    logits = jnp.dot(hm, w2_ref[...],
                     preferred_element_type=jnp.float32) + b2_ref[...]
    mx = jnp.max(logits, axis=1, keepdims=True)
    z = logits - mx
    o_ref[...] = z - jnp.log(jnp.sum(jnp.exp(z), axis=1, keepdims=True))


def _tc_final(p, batch2d, w1, b1, w2, b2):
    return pl.pallas_call(
        _final_body,
        grid=(1,),
        in_specs=[
            pl.BlockSpec((N, HQ), lambda i: (0, 0)),
            pl.BlockSpec((N, HQ), lambda i: (1, 0)),
            pl.BlockSpec((N, HQ), lambda i: (2, 0)),
            pl.BlockSpec((N, HQ), lambda i: (3, 0)),
            pl.BlockSpec((1, N), lambda i: (0, 0)),
            pl.BlockSpec((H, H), lambda i: (0, 0)),
            pl.BlockSpec((1, H), lambda i: (0, 0)),
            pl.BlockSpec((H, C), lambda i: (0, 0)),
            pl.BlockSpec((1, C), lambda i: (0, 0)),
        ],
        out_specs=pl.BlockSpec((G, C), lambda i: (0, 0)),
        out_shape=jax.ShapeDtypeStruct((G, C), jnp.float32),
    )(p, p, p, p, batch2d, w1, b1, w2, b2)


_GAT_KERNEL_CACHE = []


def _build_gat_kernel():
    mesh = plsc.VectorSubcoreMesh(core_axis_name="c", subcore_axis_name="s")

    @functools.partial(
        pl.kernel,
        out_type=jax.ShapeDtypeStruct((4 * N, HQ), jnp.float32),
        mesh=mesh,
        compiler_params=pltpu.CompilerParams(
            needs_layout_passes=False, use_tc_tiling_on_sc=False),
        scratch_types=[
            pltpu.VMEM((NR, 16), jnp.float32),
            pltpu.VMEM((NR, 16), jnp.float32),
            pltpu.VMEM((NRP, 16), jnp.float32),
            pltpu.VMEM((ET,), jnp.int32),
            pltpu.VMEM((ET,), jnp.int32),
            pltpu.VMEM((ET // 16, 16), jnp.float32),
            pltpu.VMEM((KC, HQ), jnp.float32),
            pltpu.VMEM((KC, HQ), jnp.float32),
            pltpu.VMEM((KC,), jnp.int32),
            pltpu.VMEM((KC,), jnp.int32),
            pltpu.VMEM((KC,), jnp.int32),
            pltpu.VMEM((KC,), jnp.int32),
            pltpu.VMEM((NRP,), jnp.int32),
            pltpu.VMEM((16, 16), jnp.float32),
            pltpu.VMEM_SHARED((NRP, 16), jnp.float32),
            pltpu.VMEM_SHARED((N, HQ), jnp.float32),
            pltpu.SemaphoreType.DMA,
            pltpu.SemaphoreType.DMA,
            pltpu.SemaphoreType.DMA,
            pltpu.SemaphoreType.DMA,
        ],
    )
    def k(hq0_hbm, hq1_hbm, hq2_hbm, hq3_hbm, src_hbm, dst_hbm,
          el_hbm, er_hbm, out_hbm,
          el_v, er_v, den_v, src_v, dst_v, ee_v, rows0_v, rows1_v,
          sidx0_v, sidx1_v, didx0_v, didx1_v,
          irow_v, scr_v, dens_s, outs_s, gsem0, gsem1, ssem0, ssem1):
        c = lax.axis_index("c")
        t = lax.axis_index("s")
        i16 = lax.iota(jnp.int32, 16)
        zf16 = jnp.zeros((16,), jnp.float32)

        pltpu.sync_copy(el_hbm, el_v)
        pltpu.sync_copy(er_hbm, er_v)
        eb = t * ET
        pltpu.sync_copy(src_hbm.at[pl.ds(eb, ET)], src_v)
        pltpu.sync_copy(dst_hbm.at[pl.ds(eb, ET)], dst_v)

        def zden(r, _):
            den_v[r, :] = zf16
            return 0
        lax.fori_loop(0, NRP, zden, 0)


        for g in range(NRP // 16):
            irow_v[pl.ds(16 * g, 16)] = i16 + 16 * g

        @pl.when(t == 0)
        def _():
            pltpu.sync_copy(den_v, dens_s)

        def mrow(r, carry):
            ml, mr = carry
            return (jnp.maximum(ml, el_v[r, :]), jnp.maximum(mr, er_v[r, :]))
        accl, accr = lax.fori_loop(
            0, NR, mrow,
            (jnp.full((16,), -3e38, jnp.float32),
             jnp.full((16,), -3e38, jnp.float32)))
        z16 = jnp.zeros((16,), jnp.int32)
        scr_v[0, :] = accl
        scr_v[1, :] = accr
        ml = jnp.full((16,), -3e38, jnp.float32)
        mr = jnp.full((16,), -3e38, jnp.float32)
        for j in range(16):
            jj = jnp.full((16,), j, jnp.int32)
            ml = jnp.maximum(ml, plsc.load_gather(scr_v, [z16, jj]))
            mr = jnp.maximum(mr, plsc.load_gather(scr_v, [z16 + 1, jj]))
        msum = ml + mr
        m = jnp.maximum(msum, 0.2 * msum) - 30.0

        plsc.subcore_barrier()

        def pa(g, _):
            s16 = src_v[pl.ds(g * 16, 16)]
            d16 = dst_v[pl.ds(g * 16, 16)]
            srow = lax.shift_right_logical(s16, 4)
            scol = lax.bitwise_and(s16, 15)
            drow = lax.shift_right_logical(d16, 4)
            dcol = lax.bitwise_and(d16, 15)
            elv = plsc.load_gather(el_v, [srow, scol])
            erv = plsc.load_gather(er_v, [drow, dcol])
            xv = elv + erv
            ev = jnp.where(xv > 0, xv, 0.2 * xv)
            ee = jnp.exp(ev - m)
            ee_v[g, :] = ee
            plsc.addupdate_scatter(den_v, [drow, dcol], ee)
            return 0
        lax.fori_loop(0, ET // 16, pa, 0)

        pltpu.sync_copy(den_v, dens_s.at[irow_v], add=True)
        plsc.subcore_barrier()
        pltpu.sync_copy(dens_s, den_v)

        def pb(g, _):
            ee = ee_v[g, :]
            d16 = dst_v[pl.ds(g * 16, 16)]
            drow = lax.shift_right_logical(d16, 4)
            dcol = lax.bitwise_and(d16, 15)
            dv = plsc.load_gather(den_v, [drow, dcol])
            ee_v[g, :] = ee / (dv + 1e-9)
            return 0
        lax.fori_loop(0, ET // 16, pb, 0)

        ob = t * RB
        NPAIR = ET // (2 * KC)

        def zrows(r, _):
            for j in range(HQ // 16):
                rows0_v[r, pl.ds(16 * j, 16)] = zf16
            return 0

        def fill_idx(sbuf, dbuf, off):
            for u in range(KC // 16):
                sbuf[pl.ds(16 * u, 16)] = src_v[pl.ds(off + 16 * u, 16)]
                dbuf[pl.ds(16 * u, 16)] = dst_v[pl.ds(off + 16 * u, 16)]

        def scale_rows(rbuf, off):
            def scale(i, _):
                ea = off + i
                av = plsc.load_gather(
                    ee_v, [jnp.full((16,), lax.shift_right_logical(ea, 4),
                                    jnp.int32),
                           jnp.full((16,), lax.bitwise_and(ea, 15),
                                    jnp.int32)])
                for j in range(HQ // 16):
                    blk = rbuf[i, pl.ds(16 * j, 16)]
                    rbuf[i, pl.ds(16 * j, 16)] = blk * av
                return 0
            lax.fori_loop(0, KC, scale, 0)

        for half in range(2):
            lax.fori_loop(0, KC, zrows, 0)
            for q in range(RB // KC):
                pltpu.sync_copy(rows0_v, outs_s.at[pl.ds(ob + q * KC, KC)])
            rem = RB - (RB // KC) * KC
            if rem:
                pltpu.sync_copy(rows0_v.at[pl.ds(0, rem)],
                                outs_s.at[pl.ds(ob + (RB // KC) * KC, rem)])

            @pl.when(t == NTILES - 1)
            def _():
                pltpu.sync_copy(
                    rows0_v.at[pl.ds(0, N - NTILES * RB)],
                    outs_s.at[pl.ds(NTILES * RB, N - NTILES * RB)])

            plsc.subcore_barrier()

            def start_gather(sbuf, rbuf, gsem):
                if half == 0:
                    @pl.when(c == 0)
                    def _():
                        pltpu.async_copy(hq0_hbm.at[sbuf], rbuf, gsem)

                    @pl.when(c == 1)
                    def _():
                        pltpu.async_copy(hq2_hbm.at[sbuf], rbuf, gsem)
                else:
                    @pl.when(c == 0)
                    def _():
                        pltpu.async_copy(hq1_hbm.at[sbuf], rbuf, gsem)

                    @pl.when(c == 1)
                    def _():
                        pltpu.async_copy(hq3_hbm.at[sbuf], rbuf, gsem)

            fill_idx(sidx0_v, didx0_v, 0)
            start_gather(sidx0_v, rows0_v, gsem0)

            def pc2(kk2, _):
                base0 = kk2 * (2 * KC)
                base1 = base0 + KC

                @pl.when(kk2 > 0)
                def _():
                    pltpu.make_async_copy(
                        hq0_hbm.at[sidx1_v], rows1_v, ssem1).wait()
                fill_idx(sidx1_v, didx1_v, base1)
                start_gather(sidx1_v, rows1_v, gsem1)

                pltpu.make_async_copy(
                    hq0_hbm.at[sidx0_v], rows0_v, gsem0).wait()
                scale_rows(rows0_v, base0)
                pltpu.async_copy(rows0_v, outs_s.at[didx0_v], ssem0,
                                 add=True)

                @pl.when(kk2 < NPAIR - 1)
                def _():
                    pltpu.make_async_copy(
                        hq0_hbm.at[sidx0_v], rows0_v, ssem0).wait()
                    fill_idx(sidx0_v, didx0_v, base0 + 2 * KC)
                    start_gather(sidx0_v, rows0_v, gsem0)

                pltpu.make_async_copy(
                    hq0_hbm.at[sidx1_v], rows1_v, gsem1).wait()
                scale_rows(rows1_v, base1)
                pltpu.async_copy(rows1_v, outs_s.at[didx1_v], ssem1,
                                 add=True)
                return 0
            lax.fori_loop(0, NPAIR, pc2, 0)

            pltpu.make_async_copy(hq0_hbm.at[sidx0_v], rows0_v, ssem0).wait()
            pltpu.make_async_copy(hq0_hbm.at[sidx1_v], rows1_v, ssem1).wait()
            plsc.subcore_barrier()

            qb = (2 * c + half) * N
            pltpu.sync_copy(outs_s.at[pl.ds(ob, RB)],
                            out_hbm.at[pl.ds(qb + ob, RB)])

            @pl.when(t == NTILES - 1)
            def _():
                pltpu.sync_copy(
                    outs_s.at[pl.ds(NTILES * RB, N - NTILES * RB)],
                    out_hbm.at[pl.ds(qb + NTILES * RB, N - NTILES * RB)])

            plsc.subcore_barrier()

    return k


def _gat_sc(hq0, hq1, hq2, hq3, src, dst, el2d, er2d):
    if not _GAT_KERNEL_CACHE:
        _GAT_KERNEL_CACHE.append(_build_gat_kernel())
    return _GAT_KERNEL_CACHE[0](hq0, hq1, hq2, hq3, src, dst, el2d, er2d)


def kernel(x, edge_index, batch, Wg1, al1, ar1, Wg2, al2, ar2, W1, b1, W2, b2):
    src = edge_index[0]
    dst = edge_index[1]
    aw1 = jnp.zeros((H, 8), jnp.float32).at[:, 0].set(al1).at[:, 1].set(ar1)
    aw2 = jnp.zeros((H, 8), jnp.float32).at[:, 0].set(al2).at[:, 1].set(ar2)

    h1, aux1 = _tc_entry1(x, Wg1, aw1)
    el1 = aux1[:, 0].reshape(NR, 16)
    er1 = aux1[:, 1].reshape(NR, 16)
    p1 = _gat_sc(h1[:, :HQ], h1[:, HQ:2 * HQ], h1[:, 2 * HQ:3 * HQ],
                 h1[:, 3 * HQ:], src, dst, el1, er1)

    h2, aux2 = _tc_entry2(p1, Wg2, aw2)
    el2 = aux2[:, 0].reshape(NR, 16)
    er2 = aux2[:, 1].reshape(NR, 16)
    p2 = _gat_sc(h2[:, :HQ], h2[:, HQ:2 * HQ], h2[:, 2 * HQ:3 * HQ],
                 h2[:, 3 * HQ:], src, dst, el2, er2)

    batchf = batch.astype(jnp.float32).reshape(1, N)
    return _tc_final(p2, batchf, W1,
                     b1.reshape(1, H), W2, b2.reshape(1, C))


# speedup vs baseline: 21.2425x; 1.0032x over previous
import functools

import jax
import jax.numpy as jnp
from jax import lax
from jax.experimental import pallas as pl
from jax.experimental.pallas import tpu as pltpu
from jax.experimental.pallas import tpu_sc as plsc

N = 10000
E = 320000
H = 128
HH = H // 2
HQ = H // 4
G = 64
C = 2

NCORES = 2
NTILES = 16
ET = E // NTILES
NR = N // 16
NRP = 640
KC = 80
RB = 624

_SELU_L = 1.0507009873554805
_SELU_A = 1.6732632423543772


def _selu(v):
    return _SELU_L * jnp.where(v > 0, v, _SELU_A * (jnp.exp(v) - 1.0))


def _entry1_body(x_ref, w_ref, aw_ref, h_ref, aux_ref):
    h = jnp.dot(x_ref[...], w_ref[...], preferred_element_type=jnp.float32)
    h_ref[...] = h
    aux_ref[...] = jnp.dot(h, aw_ref[...], preferred_element_type=jnp.float32)


def _entry2_body(p0_ref, p1_ref, p2_ref, p3_ref, w0_ref, w1_ref, w2_ref,
                 w3_ref, aw_ref, h_ref, aux_ref):
    h = (jnp.dot(_selu(p0_ref[...]), w0_ref[...],
                 preferred_element_type=jnp.float32)
         + jnp.dot(_selu(p1_ref[...]), w1_ref[...],
                   preferred_element_type=jnp.float32)
         + jnp.dot(_selu(p2_ref[...]), w2_ref[...],
                   preferred_element_type=jnp.float32)
         + jnp.dot(_selu(p3_ref[...]), w3_ref[...],
                   preferred_element_type=jnp.float32))
    h_ref[...] = h
    aux_ref[...] = jnp.dot(h, aw_ref[...], preferred_element_type=jnp.float32)


def _tc_entry1(x, w, aw):
    return pl.pallas_call(
        _entry1_body,
        grid=(5,),
        in_specs=[
            pl.BlockSpec((2000, H), lambda i: (i, 0)),
            pl.BlockSpec((H, H), lambda i: (0, 0)),
            pl.BlockSpec((H, 8), lambda i: (0, 0)),
        ],
        out_specs=[
            pl.BlockSpec((2000, H), lambda i: (i, 0)),
            pl.BlockSpec((2000, 8), lambda i: (i, 0)),
        ],
        out_shape=[
            jax.ShapeDtypeStruct((N, H), jnp.float32),
            jax.ShapeDtypeStruct((N, 8), jnp.float32),
        ],
    )(x, w, aw)


def _tc_entry2(p, w, aw):
    return pl.pallas_call(
        _entry2_body,
        grid=(5,),
        in_specs=[
            pl.BlockSpec((2000, HQ), lambda i: (i, 0)),
            pl.BlockSpec((2000, HQ), lambda i: (i + 5, 0)),
            pl.BlockSpec((2000, HQ), lambda i: (i + 10, 0)),
            pl.BlockSpec((2000, HQ), lambda i: (i + 15, 0)),
            pl.BlockSpec((HQ, H), lambda i: (0, 0)),
            pl.BlockSpec((HQ, H), lambda i: (1, 0)),
            pl.BlockSpec((HQ, H), lambda i: (2, 0)),
            pl.BlockSpec((HQ, H), lambda i: (3, 0)),
            pl.BlockSpec((H, 8), lambda i: (0, 0)),
        ],
        out_specs=[
            pl.BlockSpec((2000, H), lambda i: (i, 0)),
            pl.BlockSpec((2000, 8), lambda i: (i, 0)),
        ],
        out_shape=[
            jax.ShapeDtypeStruct((N, H), jnp.float32),
            jax.ShapeDtypeStruct((N, 8), jnp.float32),
        ],
    )(p, p, p, p, w, w, w, w, aw)


def _final_body(p0_ref, p1_ref, p2_ref, p3_ref, b_ref, w1_ref, b1_ref,
                w2_ref, b2_ref, o_ref):
    a = jnp.concatenate([_selu(p0_ref[...]), _selu(p1_ref[...]),
                         _selu(p2_ref[...]), _selu(p3_ref[...])], axis=1)
    bt = b_ref[...]
    gid = lax.broadcasted_iota(jnp.int32, (G, N), 0).astype(jnp.float32)
    P = jnp.where(gid == bt, 1.0, 0.0).astype(jnp.float32)
    cnt = jnp.sum(P, axis=1, keepdims=True)
    pooled = jnp.dot(P, a, preferred_element_type=jnp.float32)
    pooled = pooled / jnp.maximum(cnt, 1.0)
    hm = _selu(jnp.dot(pooled, w1_ref[...],
                       preferred_element_type=jnp.float32) + b1_ref[...])
    logits = jnp.dot(hm, w2_ref[...],
                     preferred_element_type=jnp.float32) + b2_ref[...]
    mx = jnp.max(logits, axis=1, keepdims=True)
    z = logits - mx
    o_ref[...] = z - jnp.log(jnp.sum(jnp.exp(z), axis=1, keepdims=True))


def _tc_final(p, batch2d, w1, b1, w2, b2):
    return pl.pallas_call(
        _final_body,
        grid=(1,),
        in_specs=[
            pl.BlockSpec((N, HQ), lambda i: (0, 0)),
            pl.BlockSpec((N, HQ), lambda i: (1, 0)),
            pl.BlockSpec((N, HQ), lambda i: (2, 0)),
            pl.BlockSpec((N, HQ), lambda i: (3, 0)),
            pl.BlockSpec((1, N), lambda i: (0, 0)),
            pl.BlockSpec((H, H), lambda i: (0, 0)),
            pl.BlockSpec((1, H), lambda i: (0, 0)),
            pl.BlockSpec((H, C), lambda i: (0, 0)),
            pl.BlockSpec((1, C), lambda i: (0, 0)),
        ],
        out_specs=pl.BlockSpec((G, C), lambda i: (0, 0)),
        out_shape=jax.ShapeDtypeStruct((G, C), jnp.float32),
    )(p, p, p, p, batch2d, w1, b1, w2, b2)


_GAT_KERNEL_CACHE = []


def _build_gat_kernel():
    mesh = plsc.VectorSubcoreMesh(core_axis_name="c", subcore_axis_name="s")

    @functools.partial(
        pl.kernel,
        out_type=jax.ShapeDtypeStruct((4 * N, HQ), jnp.float32),
        mesh=mesh,
        compiler_params=pltpu.CompilerParams(
            needs_layout_passes=False, use_tc_tiling_on_sc=False),
        scratch_types=[
            pltpu.VMEM((NR, 16), jnp.float32),
            pltpu.VMEM((NR, 16), jnp.float32),
            pltpu.VMEM((NRP, 16), jnp.float32),
            pltpu.VMEM((ET,), jnp.int32),
            pltpu.VMEM((ET,), jnp.int32),
            pltpu.VMEM((ET // 16, 16), jnp.float32),
            pltpu.VMEM((KC, HQ), jnp.float32),
            pltpu.VMEM((KC, HQ), jnp.float32),
            pltpu.VMEM((KC,), jnp.int32),
            pltpu.VMEM((KC,), jnp.int32),
            pltpu.VMEM((KC,), jnp.int32),
            pltpu.VMEM((KC,), jnp.int32),
            pltpu.VMEM((NRP,), jnp.int32),
            pltpu.VMEM((16, 16), jnp.float32),
            pltpu.VMEM_SHARED((NRP, 16), jnp.float32),
            pltpu.VMEM_SHARED((N, HQ), jnp.float32),
            pltpu.SemaphoreType.DMA,
            pltpu.SemaphoreType.DMA,
            pltpu.SemaphoreType.DMA,
            pltpu.SemaphoreType.DMA,
        ],
    )
    def k(hq0_hbm, hq1_hbm, hq2_hbm, hq3_hbm, src_hbm, dst_hbm,
          el_hbm, er_hbm, out_hbm,
          el_v, er_v, den_v, src_v, dst_v, ee_v, rows0_v, rows1_v,
          sidx0_v, sidx1_v, didx0_v, didx1_v,
          irow_v, scr_v, dens_s, outs_s, gsem0, gsem1, ssem0, ssem1):
        c = lax.axis_index("c")
        t = lax.axis_index("s")
        i16 = lax.iota(jnp.int32, 16)
        zf16 = jnp.zeros((16,), jnp.float32)

        pltpu.sync_copy(el_hbm, el_v)
        pltpu.sync_copy(er_hbm, er_v)
        eb = t * ET
        pltpu.sync_copy(src_hbm.at[pl.ds(eb, ET)], src_v)
        pltpu.sync_copy(dst_hbm.at[pl.ds(eb, ET)], dst_v)

        def zden(r, _):
            den_v[r, :] = zf16
            return 0
        lax.fori_loop(0, NRP, zden, 0)


        for g in range(NRP // 16):
            irow_v[pl.ds(16 * g, 16)] = i16 + 16 * g

        @pl.when(t == 0)
        def _():
            pltpu.sync_copy(den_v, dens_s)

        def mrow(r, carry):
            ml, mr = carry
            return (jnp.maximum(ml, el_v[r, :]), jnp.maximum(mr, er_v[r, :]))
        accl, accr = lax.fori_loop(
            0, NR, mrow,
            (jnp.full((16,), -3e38, jnp.float32),
             jnp.full((16,), -3e38, jnp.float32)))
        z16 = jnp.zeros((16,), jnp.int32)
        scr_v[0, :] = accl
        scr_v[1, :] = accr
        ml = jnp.full((16,), -3e38, jnp.float32)
        mr = jnp.full((16,), -3e38, jnp.float32)
        for j in range(16):
            jj = jnp.full((16,), j, jnp.int32)
            ml = jnp.maximum(ml, plsc.load_gather(scr_v, [z16, jj]))
            mr = jnp.maximum(mr, plsc.load_gather(scr_v, [z16 + 1, jj]))
        msum = ml + mr
        m = jnp.maximum(msum, 0.2 * msum) - 30.0

        plsc.subcore_barrier()

        def pa(g, _):
            s16 = src_v[pl.ds(g * 16, 16)]
            d16 = dst_v[pl.ds(g * 16, 16)]
            srow = lax.shift_right_logical(s16, 4)
            scol = lax.bitwise_and(s16, 15)
            drow = lax.shift_right_logical(d16, 4)
            dcol = lax.bitwise_and(d16, 15)
            elv = plsc.load_gather(el_v, [srow, scol])
            erv = plsc.load_gather(er_v, [drow, dcol])
            xv = elv + erv
            ev = jnp.where(xv > 0, xv, 0.2 * xv)
            ee = jnp.exp(ev - m)
            ee_v[g, :] = ee
            plsc.addupdate_scatter(den_v, [drow, dcol], ee)
            return 0
        lax.fori_loop(0, ET // 16, pa, 0)

        pltpu.sync_copy(den_v, dens_s.at[irow_v], add=True)
        plsc.subcore_barrier()
        pltpu.sync_copy(dens_s, den_v)

        def pb(g, _):
            ee = ee_v[g, :]
            d16 = dst_v[pl.ds(g * 16, 16)]
            drow = lax.shift_right_logical(d16, 4)
            dcol = lax.bitwise_and(d16, 15)
            dv = plsc.load_gather(den_v, [drow, dcol])
            ee_v[g, :] = ee / (dv + 1e-9)
            return 0
        lax.fori_loop(0, ET // 16, pb, 0)

        ob = t * RB
        NPAIR = ET // (2 * KC)

        def zrows(r, _):
            for j in range(HQ // 16):
                rows0_v[r, pl.ds(16 * j, 16)] = zf16
            return 0

        def fill_idx(sbuf, dbuf, off):
            for u in range(KC // 16):
                sbuf[pl.ds(16 * u, 16)] = src_v[pl.ds(off + 16 * u, 16)]
                dbuf[pl.ds(16 * u, 16)] = dst_v[pl.ds(off + 16 * u, 16)]

        def scale_rows(rbuf, off):
            def scale(i4, _):
                for s in range(4):
                    i = i4 * 4 + s
                    ea = off + i
                    av = plsc.load_gather(
                        ee_v, [jnp.full((16,),
                                        lax.shift_right_logical(ea, 4),
                                        jnp.int32),
                               jnp.full((16,), lax.bitwise_and(ea, 15),
                                        jnp.int32)])
                    for j in range(HQ // 16):
                        blk = rbuf[i, pl.ds(16 * j, 16)]
                        rbuf[i, pl.ds(16 * j, 16)] = blk * av
                return 0
            lax.fori_loop(0, KC // 4, scale, 0)

        for half in range(2):
            lax.fori_loop(0, KC, zrows, 0)
            nzf = 0
            for q in range(RB // KC):
                pltpu.async_copy(rows0_v, outs_s.at[pl.ds(ob + q * KC, KC)],
                                 gsem0)
                nzf += 1
            rem = RB - (RB // KC) * KC
            if rem:
                pltpu.async_copy(rows0_v.at[pl.ds(0, rem)],
                                 outs_s.at[pl.ds(ob + (RB // KC) * KC, rem)],
                                 gsem1)

            @pl.when(t == NTILES - 1)
            def _():
                pltpu.async_copy(
                    rows0_v.at[pl.ds(0, N - NTILES * RB)],
                    outs_s.at[pl.ds(NTILES * RB, N - NTILES * RB)], ssem0)

            for q in range(nzf):
                pltpu.make_async_copy(
                    rows0_v, outs_s.at[pl.ds(ob, KC)], gsem0).wait()
            if rem:
                pltpu.make_async_copy(
                    rows0_v.at[pl.ds(0, rem)],
                    outs_s.at[pl.ds(ob, rem)], gsem1).wait()

            @pl.when(t == NTILES - 1)
            def _():
                pltpu.make_async_copy(
                    rows0_v.at[pl.ds(0, N - NTILES * RB)],
                    outs_s.at[pl.ds(NTILES * RB, N - NTILES * RB)],
                    ssem0).wait()

            plsc.subcore_barrier()

            def start_gather(sbuf, rbuf, gsem):
                if half == 0:
                    @pl.when(c == 0)
                    def _():
                        pltpu.async_copy(hq0_hbm.at[sbuf], rbuf, gsem)

                    @pl.when(c == 1)
                    def _():
                        pltpu.async_copy(hq2_hbm.at[sbuf], rbuf, gsem)
                else:
                    @pl.when(c == 0)
                    def _():
                        pltpu.async_copy(hq1_hbm.at[sbuf], rbuf, gsem)

                    @pl.when(c == 1)
                    def _():
                        pltpu.async_copy(hq3_hbm.at[sbuf], rbuf, gsem)

            fill_idx(sidx0_v, didx0_v, 0)
            start_gather(sidx0_v, rows0_v, gsem0)

            def pc2(kk2, _):
                base0 = kk2 * (2 * KC)
                base1 = base0 + KC

                @pl.when(kk2 > 0)
                def _():
                    pltpu.make_async_copy(
                        hq0_hbm.at[sidx1_v], rows1_v, ssem1).wait()
                fill_idx(sidx1_v, didx1_v, base1)
                start_gather(sidx1_v, rows1_v, gsem1)

                pltpu.make_async_copy(
                    hq0_hbm.at[sidx0_v], rows0_v, gsem0).wait()
                scale_rows(rows0_v, base0)
                pltpu.async_copy(rows0_v, outs_s.at[didx0_v], ssem0,
                                 add=True)

                @pl.when(kk2 < NPAIR - 1)
                def _():
                    pltpu.make_async_copy(
                        hq0_hbm.at[sidx0_v], rows0_v, ssem0).wait()
                    fill_idx(sidx0_v, didx0_v, base0 + 2 * KC)
                    start_gather(sidx0_v, rows0_v, gsem0)

                pltpu.make_async_copy(
                    hq0_hbm.at[sidx1_v], rows1_v, gsem1).wait()
                scale_rows(rows1_v, base1)
                pltpu.async_copy(rows1_v, outs_s.at[didx1_v], ssem1,
                                 add=True)
                return 0
            lax.fori_loop(0, NPAIR, pc2, 0)

            pltpu.make_async_copy(hq0_hbm.at[sidx0_v], rows0_v, ssem0).wait()
            pltpu.make_async_copy(hq0_hbm.at[sidx1_v], rows1_v, ssem1).wait()
            plsc.subcore_barrier()

            qb = (2 * c + half) * N
            pltpu.sync_copy(outs_s.at[pl.ds(ob, RB)],
                            out_hbm.at[pl.ds(qb + ob, RB)])

            @pl.when(t == NTILES - 1)
            def _():
                pltpu.sync_copy(
                    outs_s.at[pl.ds(NTILES * RB, N - NTILES * RB)],
                    out_hbm.at[pl.ds(qb + NTILES * RB, N - NTILES * RB)])

            plsc.subcore_barrier()

    return k


def _gat_sc(hq0, hq1, hq2, hq3, src, dst, el2d, er2d):
    if not _GAT_KERNEL_CACHE:
        _GAT_KERNEL_CACHE.append(_build_gat_kernel())
    return _GAT_KERNEL_CACHE[0](hq0, hq1, hq2, hq3, src, dst, el2d, er2d)


def kernel(x, edge_index, batch, Wg1, al1, ar1, Wg2, al2, ar2, W1, b1, W2, b2):
    src = edge_index[0]
    dst = edge_index[1]
    aw1 = jnp.zeros((H, 8), jnp.float32).at[:, 0].set(al1).at[:, 1].set(ar1)
    aw2 = jnp.zeros((H, 8), jnp.float32).at[:, 0].set(al2).at[:, 1].set(ar2)

    h1, aux1 = _tc_entry1(x, Wg1, aw1)
    el1 = aux1[:, 0].reshape(NR, 16)
    er1 = aux1[:, 1].reshape(NR, 16)
    p1 = _gat_sc(h1[:, :HQ], h1[:, HQ:2 * HQ], h1[:, 2 * HQ:3 * HQ],
                 h1[:, 3 * HQ:], src, dst, el1, er1)

    h2, aux2 = _tc_entry2(p1, Wg2, aw2)
    el2 = aux2[:, 0].reshape(NR, 16)
    er2 = aux2[:, 1].reshape(NR, 16)
    p2 = _gat_sc(h2[:, :HQ], h2[:, HQ:2 * HQ], h2[:, 2 * HQ:3 * HQ],
                 h2[:, 3 * HQ:], src, dst, el2, er2)

    batchf = batch.astype(jnp.float32).reshape(1, N)
    return _tc_final(p2, batchf, W1,
                     b1.reshape(1, H), W2, b2.reshape(1, C))


# speedup vs baseline: 21.6462x; 1.0190x over previous
import functools

import jax
import jax.numpy as jnp
from jax import lax
from jax.experimental import pallas as pl
from jax.experimental.pallas import tpu as pltpu
from jax.experimental.pallas import tpu_sc as plsc

N = 10000
E = 320000
H = 128
HH = H // 2
HQ = H // 4
G = 64
C = 2

NCORES = 2
NTILES = 16
ET = E // NTILES
NR = N // 16
NRP = 640
KC = 80
RB = 624

_SELU_L = 1.0507009873554805
_SELU_A = 1.6732632423543772


def _selu(v):
    return _SELU_L * jnp.where(v > 0, v, _SELU_A * (jnp.exp(v) - 1.0))


def _entry1_body(x_ref, w_ref, aw_ref, q0_ref, q1_ref, q2_ref, q3_ref,
                 aux_ref):
    h = jnp.dot(x_ref[...], w_ref[...], preferred_element_type=jnp.float32)
    q0_ref[...] = h[:, :HQ]
    q1_ref[...] = h[:, HQ:2 * HQ]
    q2_ref[...] = h[:, 2 * HQ:3 * HQ]
    q3_ref[...] = h[:, 3 * HQ:]
    aux_ref[...] = jnp.dot(h, aw_ref[...], preferred_element_type=jnp.float32)


def _entry2_body(p0_ref, p1_ref, p2_ref, p3_ref, w0_ref, w1_ref, w2_ref,
                 w3_ref, aw_ref, q0_ref, q1_ref, q2_ref, q3_ref, aux_ref):
    h = (jnp.dot(_selu(p0_ref[...]), w0_ref[...],
                 preferred_element_type=jnp.float32)
         + jnp.dot(_selu(p1_ref[...]), w1_ref[...],
                   preferred_element_type=jnp.float32)
         + jnp.dot(_selu(p2_ref[...]), w2_ref[...],
                   preferred_element_type=jnp.float32)
         + jnp.dot(_selu(p3_ref[...]), w3_ref[...],
                   preferred_element_type=jnp.float32))
    q0_ref[...] = h[:, :HQ]
    q1_ref[...] = h[:, HQ:2 * HQ]
    q2_ref[...] = h[:, 2 * HQ:3 * HQ]
    q3_ref[...] = h[:, 3 * HQ:]
    aux_ref[...] = jnp.dot(h, aw_ref[...], preferred_element_type=jnp.float32)


def _tc_entry1(x, w, aw):
    return pl.pallas_call(
        _entry1_body,
        grid=(5,),
        in_specs=[
            pl.BlockSpec((2000, H), lambda i: (i, 0)),
            pl.BlockSpec((H, H), lambda i: (0, 0)),
            pl.BlockSpec((H, 8), lambda i: (0, 0)),
        ],
        out_specs=[
            pl.BlockSpec((2000, HQ), lambda i: (i, 0)),
            pl.BlockSpec((2000, HQ), lambda i: (i, 0)),
            pl.BlockSpec((2000, HQ), lambda i: (i, 0)),
            pl.BlockSpec((2000, HQ), lambda i: (i, 0)),
            pl.BlockSpec((2000, 8), lambda i: (i, 0)),
        ],
        out_shape=[
            jax.ShapeDtypeStruct((N, HQ), jnp.float32),
            jax.ShapeDtypeStruct((N, HQ), jnp.float32),
            jax.ShapeDtypeStruct((N, HQ), jnp.float32),
            jax.ShapeDtypeStruct((N, HQ), jnp.float32),
            jax.ShapeDtypeStruct((N, 8), jnp.float32),
        ],
    )(x, w, aw)


def _tc_entry2(p, w, aw):
    return pl.pallas_call(
        _entry2_body,
        grid=(5,),
        in_specs=[
            pl.BlockSpec((2000, HQ), lambda i: (i, 0)),
            pl.BlockSpec((2000, HQ), lambda i: (i + 5, 0)),
            pl.BlockSpec((2000, HQ), lambda i: (i + 10, 0)),
            pl.BlockSpec((2000, HQ), lambda i: (i + 15, 0)),
            pl.BlockSpec((HQ, H), lambda i: (0, 0)),
            pl.BlockSpec((HQ, H), lambda i: (1, 0)),
            pl.BlockSpec((HQ, H), lambda i: (2, 0)),
            pl.BlockSpec((HQ, H), lambda i: (3, 0)),
            pl.BlockSpec((H, 8), lambda i: (0, 0)),
        ],
        out_specs=[
            pl.BlockSpec((2000, HQ), lambda i: (i, 0)),
            pl.BlockSpec((2000, HQ), lambda i: (i, 0)),
            pl.BlockSpec((2000, HQ), lambda i: (i, 0)),
            pl.BlockSpec((2000, HQ), lambda i: (i, 0)),
            pl.BlockSpec((2000, 8), lambda i: (i, 0)),
        ],
        out_shape=[
            jax.ShapeDtypeStruct((N, HQ), jnp.float32),
            jax.ShapeDtypeStruct((N, HQ), jnp.float32),
            jax.ShapeDtypeStruct((N, HQ), jnp.float32),
            jax.ShapeDtypeStruct((N, HQ), jnp.float32),
            jax.ShapeDtypeStruct((N, 8), jnp.float32),
        ],
    )(p, p, p, p, w, w, w, w, aw)


def _final_body(p0_ref, p1_ref, p2_ref, p3_ref, b_ref, w1_ref, b1_ref,
                w2_ref, b2_ref, o_ref):
    a = jnp.concatenate([_selu(p0_ref[...]), _selu(p1_ref[...]),
                         _selu(p2_ref[...]), _selu(p3_ref[...])], axis=1)
    bt = b_ref[...]
    gid = lax.broadcasted_iota(jnp.int32, (G, N), 0).astype(jnp.float32)
    P = jnp.where(gid == bt, 1.0, 0.0).astype(jnp.float32)
    cnt = jnp.sum(P, axis=1, keepdims=True)
    pooled = jnp.dot(P, a, preferred_element_type=jnp.float32)
    pooled = pooled / jnp.maximum(cnt, 1.0)
    hm = _selu(jnp.dot(pooled, w1_ref[...],
                       preferred_element_type=jnp.float32) + b1_ref[...])
    logits = jnp.dot(hm, w2_ref[...],
                     preferred_element_type=jnp.float32) + b2_ref[...]
    mx = jnp.max(logits, axis=1, keepdims=True)
    z = logits - mx
    o_ref[...] = z - jnp.log(jnp.sum(jnp.exp(z), axis=1, keepdims=True))


def _tc_final(p, batch2d, w1, b1, w2, b2):
    return pl.pallas_call(
        _final_body,
        grid=(1,),
        in_specs=[
            pl.BlockSpec((N, HQ), lambda i: (0, 0)),
            pl.BlockSpec((N, HQ), lambda i: (1, 0)),
            pl.BlockSpec((N, HQ), lambda i: (2, 0)),
            pl.BlockSpec((N, HQ), lambda i: (3, 0)),
            pl.BlockSpec((1, N), lambda i: (0, 0)),
            pl.BlockSpec((H, H), lambda i: (0, 0)),
            pl.BlockSpec((1, H), lambda i: (0, 0)),
            pl.BlockSpec((H, C), lambda i: (0, 0)),
            pl.BlockSpec((1, C), lambda i: (0, 0)),
        ],
        out_specs=pl.BlockSpec((G, C), lambda i: (0, 0)),
        out_shape=jax.ShapeDtypeStruct((G, C), jnp.float32),
    )(p, p, p, p, batch2d, w1, b1, w2, b2)


_GAT_KERNEL_CACHE = []


def _build_gat_kernel():
    mesh = plsc.VectorSubcoreMesh(core_axis_name="c", subcore_axis_name="s")

    @functools.partial(
        pl.kernel,
        out_type=jax.ShapeDtypeStruct((4 * N, HQ), jnp.float32),
        mesh=mesh,
        compiler_params=pltpu.CompilerParams(
            needs_layout_passes=False, use_tc_tiling_on_sc=False),
        scratch_types=[
            pltpu.VMEM((NR, 16), jnp.float32),
            pltpu.VMEM((NR, 16), jnp.float32),
            pltpu.VMEM((NRP, 16), jnp.float32),
            pltpu.VMEM((ET,), jnp.int32),
            pltpu.VMEM((ET,), jnp.int32),
            pltpu.VMEM((ET // 16, 16), jnp.float32),
            pltpu.VMEM((KC, HQ), jnp.float32),
            pltpu.VMEM((KC, HQ), jnp.float32),
            pltpu.VMEM((KC,), jnp.int32),
            pltpu.VMEM((KC,), jnp.int32),
            pltpu.VMEM((KC,), jnp.int32),
            pltpu.VMEM((KC,), jnp.int32),
            pltpu.VMEM((NRP,), jnp.int32),
            pltpu.VMEM((16, 16), jnp.float32),
            pltpu.VMEM_SHARED((NRP, 16), jnp.float32),
            pltpu.VMEM_SHARED((N, HQ), jnp.float32),
            pltpu.SemaphoreType.DMA,
            pltpu.SemaphoreType.DMA,
            pltpu.SemaphoreType.DMA,
            pltpu.SemaphoreType.DMA,
        ],
    )
    def k(hq0_hbm, hq1_hbm, hq2_hbm, hq3_hbm, src_hbm, dst_hbm,
          el_hbm, er_hbm, out_hbm,
          el_v, er_v, den_v, src_v, dst_v, ee_v, rows0_v, rows1_v,
          sidx0_v, sidx1_v, didx0_v, didx1_v,
          irow_v, scr_v, dens_s, outs_s, gsem0, gsem1, ssem0, ssem1):
        c = lax.axis_index("c")
        t = lax.axis_index("s")
        i16 = lax.iota(jnp.int32, 16)
        zf16 = jnp.zeros((16,), jnp.float32)

        pltpu.sync_copy(el_hbm, el_v)
        pltpu.sync_copy(er_hbm, er_v)
        eb = t * ET
        pltpu.sync_copy(src_hbm.at[pl.ds(eb, ET)], src_v)
        pltpu.sync_copy(dst_hbm.at[pl.ds(eb, ET)], dst_v)

        def zden(r, _):
            den_v[r, :] = zf16
            return 0
        lax.fori_loop(0, NRP, zden, 0)


        for g in range(NRP // 16):
            irow_v[pl.ds(16 * g, 16)] = i16 + 16 * g

        @pl.when(t == 0)
        def _():
            pltpu.sync_copy(den_v, dens_s)

        def mrow(r, carry):
            ml, mr = carry
            return (jnp.maximum(ml, el_v[r, :]), jnp.maximum(mr, er_v[r, :]))
        accl, accr = lax.fori_loop(
            0, NR, mrow,
            (jnp.full((16,), -3e38, jnp.float32),
             jnp.full((16,), -3e38, jnp.float32)))
        z16 = jnp.zeros((16,), jnp.int32)
        scr_v[0, :] = accl
        scr_v[1, :] = accr
        ml = jnp.full((16,), -3e38, jnp.float32)
        mr = jnp.full((16,), -3e38, jnp.float32)
        for j in range(16):
            jj = jnp.full((16,), j, jnp.int32)
            ml = jnp.maximum(ml, plsc.load_gather(scr_v, [z16, jj]))
            mr = jnp.maximum(mr, plsc.load_gather(scr_v, [z16 + 1, jj]))
        msum = ml + mr
        m = jnp.maximum(msum, 0.2 * msum) - 30.0

        plsc.subcore_barrier()

        def pa(g, _):
            s16 = src_v[pl.ds(g * 16, 16)]
            d16 = dst_v[pl.ds(g * 16, 16)]
            srow = lax.shift_right_logical(s16, 4)
            scol = lax.bitwise_and(s16, 15)
            drow = lax.shift_right_logical(d16, 4)
            dcol = lax.bitwise_and(d16, 15)
            elv = plsc.load_gather(el_v, [srow, scol])
            erv = plsc.load_gather(er_v, [drow, dcol])
            xv = elv + erv
            ev = jnp.where(xv > 0, xv, 0.2 * xv)
            ee = jnp.exp(ev - m)
            ee_v[g, :] = ee
            plsc.addupdate_scatter(den_v, [drow, dcol], ee)
            return 0
        lax.fori_loop(0, ET // 16, pa, 0)

        pltpu.sync_copy(den_v, dens_s.at[irow_v], add=True)
        plsc.subcore_barrier()
        pltpu.sync_copy(dens_s, den_v)

        def pb(g, _):
            ee = ee_v[g, :]
            d16 = dst_v[pl.ds(g * 16, 16)]
            drow = lax.shift_right_logical(d16, 4)
            dcol = lax.bitwise_and(d16, 15)
            dv = plsc.load_gather(den_v, [drow, dcol])
            ee_v[g, :] = ee / (dv + 1e-9)
            return 0
        lax.fori_loop(0, ET // 16, pb, 0)

        ob = t * RB
        NPAIR = ET // (2 * KC)

        def zrows(r, _):
            for j in range(HQ // 16):
                rows0_v[r, pl.ds(16 * j, 16)] = zf16
            return 0

        def fill_idx(sbuf, dbuf, off):
            for u in range(KC // 16):
                sbuf[pl.ds(16 * u, 16)] = src_v[pl.ds(off + 16 * u, 16)]
                dbuf[pl.ds(16 * u, 16)] = dst_v[pl.ds(off + 16 * u, 16)]

        def scale_rows(rbuf, off):
            def scale(i4, _):
                for s in range(4):
                    i = i4 * 4 + s
                    ea = off + i
                    av = plsc.load_gather(
                        ee_v, [jnp.full((16,),
                                        lax.shift_right_logical(ea, 4),
                                        jnp.int32),
                               jnp.full((16,), lax.bitwise_and(ea, 15),
                                        jnp.int32)])
                    for j in range(HQ // 16):
                        blk = rbuf[i, pl.ds(16 * j, 16)]
                        rbuf[i, pl.ds(16 * j, 16)] = blk * av
                return 0
            lax.fori_loop(0, KC // 4, scale, 0)

        for half in range(2):
            lax.fori_loop(0, KC, zrows, 0)
            nzf = 0
            for q in range(RB // KC):
                pltpu.async_copy(rows0_v, outs_s.at[pl.ds(ob + q * KC, KC)],
                                 gsem0)
                nzf += 1
            rem = RB - (RB // KC) * KC
            if rem:
                pltpu.async_copy(rows0_v.at[pl.ds(0, rem)],
                                 outs_s.at[pl.ds(ob + (RB // KC) * KC, rem)],
                                 gsem1)

            @pl.when(t == NTILES - 1)
            def _():
                pltpu.async_copy(
                    rows0_v.at[pl.ds(0, N - NTILES * RB)],
                    outs_s.at[pl.ds(NTILES * RB, N - NTILES * RB)], ssem0)

            for q in range(nzf):
                pltpu.make_async_copy(
                    rows0_v, outs_s.at[pl.ds(ob, KC)], gsem0).wait()
            if rem:
                pltpu.make_async_copy(
                    rows0_v.at[pl.ds(0, rem)],
                    outs_s.at[pl.ds(ob, rem)], gsem1).wait()

            @pl.when(t == NTILES - 1)
            def _():
                pltpu.make_async_copy(
                    rows0_v.at[pl.ds(0, N - NTILES * RB)],
                    outs_s.at[pl.ds(NTILES * RB, N - NTILES * RB)],
                    ssem0).wait()

            plsc.subcore_barrier()

            def start_gather(sbuf, rbuf, gsem):
                if half == 0:
                    @pl.when(c == 0)
                    def _():
                        pltpu.async_copy(hq0_hbm.at[sbuf], rbuf, gsem)

                    @pl.when(c == 1)
                    def _():
                        pltpu.async_copy(hq2_hbm.at[sbuf], rbuf, gsem)
                else:
                    @pl.when(c == 0)
                    def _():
                        pltpu.async_copy(hq1_hbm.at[sbuf], rbuf, gsem)

                    @pl.when(c == 1)
                    def _():
                        pltpu.async_copy(hq3_hbm.at[sbuf], rbuf, gsem)

            fill_idx(sidx0_v, didx0_v, 0)
            start_gather(sidx0_v, rows0_v, gsem0)

            def pc2(kk2, _):
                base0 = kk2 * (2 * KC)
                base1 = base0 + KC

                @pl.when(kk2 > 0)
                def _():
                    pltpu.make_async_copy(
                        hq0_hbm.at[sidx1_v], rows1_v, ssem1).wait()
                fill_idx(sidx1_v, didx1_v, base1)
                start_gather(sidx1_v, rows1_v, gsem1)

                pltpu.make_async_copy(
                    hq0_hbm.at[sidx0_v], rows0_v, gsem0).wait()
                scale_rows(rows0_v, base0)
                pltpu.async_copy(rows0_v, outs_s.at[didx0_v], ssem0,
                                 add=True)

                @pl.when(kk2 < NPAIR - 1)
                def _():
                    pltpu.make_async_copy(
                        hq0_hbm.at[sidx0_v], rows0_v, ssem0).wait()
                    fill_idx(sidx0_v, didx0_v, base0 + 2 * KC)
                    start_gather(sidx0_v, rows0_v, gsem0)

                pltpu.make_async_copy(
                    hq0_hbm.at[sidx1_v], rows1_v, gsem1).wait()
                scale_rows(rows1_v, base1)
                pltpu.async_copy(rows1_v, outs_s.at[didx1_v], ssem1,
                                 add=True)
                return 0
            lax.fori_loop(0, NPAIR, pc2, 0)

            pltpu.make_async_copy(hq0_hbm.at[sidx0_v], rows0_v, ssem0).wait()
            pltpu.make_async_copy(hq0_hbm.at[sidx1_v], rows1_v, ssem1).wait()
            plsc.subcore_barrier()

            qb = (2 * c + half) * N
            pltpu.sync_copy(outs_s.at[pl.ds(ob, RB)],
                            out_hbm.at[pl.ds(qb + ob, RB)])

            @pl.when(t == NTILES - 1)
            def _():
                pltpu.sync_copy(
                    outs_s.at[pl.ds(NTILES * RB, N - NTILES * RB)],
                    out_hbm.at[pl.ds(qb + NTILES * RB, N - NTILES * RB)])

            plsc.subcore_barrier()

    return k


def _gat_sc(hq0, hq1, hq2, hq3, src, dst, el2d, er2d):
    if not _GAT_KERNEL_CACHE:
        _GAT_KERNEL_CACHE.append(_build_gat_kernel())
    return _GAT_KERNEL_CACHE[0](hq0, hq1, hq2, hq3, src, dst, el2d, er2d)


def kernel(x, edge_index, batch, Wg1, al1, ar1, Wg2, al2, ar2, W1, b1, W2, b2):
    src = edge_index[0]
    dst = edge_index[1]
    zcol = jnp.zeros((H,), jnp.float32)
    aw1 = jnp.stack([al1, ar1, zcol, zcol, zcol, zcol, zcol, zcol], axis=1)
    aw2 = jnp.stack([al2, ar2, zcol, zcol, zcol, zcol, zcol, zcol], axis=1)

    q10, q11, q12, q13, aux1 = _tc_entry1(x, Wg1, aw1)
    el1 = aux1[:, 0].reshape(NR, 16)
    er1 = aux1[:, 1].reshape(NR, 16)
    p1 = _gat_sc(q10, q11, q12, q13, src, dst, el1, er1)

    q20, q21, q22, q23, aux2 = _tc_entry2(p1, Wg2, aw2)
    el2 = aux2[:, 0].reshape(NR, 16)
    er2 = aux2[:, 1].reshape(NR, 16)
    p2 = _gat_sc(q20, q21, q22, q23, src, dst, el2, er2)

    batchf = batch.astype(jnp.float32).reshape(1, N)
    return _tc_final(p2, batchf, W1,
                     b1.reshape(1, H), W2, b2.reshape(1, C))


# speedup vs baseline: 22.9024x; 1.0580x over previous
import functools

import jax
import jax.numpy as jnp
from jax import lax
from jax.experimental import pallas as pl
from jax.experimental.pallas import tpu as pltpu
from jax.experimental.pallas import tpu_sc as plsc

N = 10000
E = 320000
H = 128
HH = H // 2
HQ = H // 4
G = 64
C = 2

NCORES = 2
NTILES = 16
ET = E // NTILES
NR = N // 16
NRP = 640
KC = 128
KT = ET - (ET // (2 * KC)) * 2 * KC
RB = 624

_SELU_L = 1.0507009873554805
_SELU_A = 1.6732632423543772


def _selu(v):
    return _SELU_L * jnp.where(v > 0, v, _SELU_A * (jnp.exp(v) - 1.0))


def _entry1_body(x_ref, w_ref, aw_ref, q0_ref, q1_ref, q2_ref, q3_ref,
                 aux_ref):
    h = jnp.dot(x_ref[...], w_ref[...], preferred_element_type=jnp.float32)
    q0_ref[...] = h[:, :HQ]
    q1_ref[...] = h[:, HQ:2 * HQ]
    q2_ref[...] = h[:, 2 * HQ:3 * HQ]
    q3_ref[...] = h[:, 3 * HQ:]
    aux_ref[...] = jnp.dot(h, aw_ref[...], preferred_element_type=jnp.float32)


def _entry2_body(p0_ref, p1_ref, p2_ref, p3_ref, w0_ref, w1_ref, w2_ref,
                 w3_ref, aw_ref, q0_ref, q1_ref, q2_ref, q3_ref, aux_ref):
    h = (jnp.dot(_selu(p0_ref[...]), w0_ref[...],
                 preferred_element_type=jnp.float32)
         + jnp.dot(_selu(p1_ref[...]), w1_ref[...],
                   preferred_element_type=jnp.float32)
         + jnp.dot(_selu(p2_ref[...]), w2_ref[...],
                   preferred_element_type=jnp.float32)
         + jnp.dot(_selu(p3_ref[...]), w3_ref[...],
                   preferred_element_type=jnp.float32))
    q0_ref[...] = h[:, :HQ]
    q1_ref[...] = h[:, HQ:2 * HQ]
    q2_ref[...] = h[:, 2 * HQ:3 * HQ]
    q3_ref[...] = h[:, 3 * HQ:]
    aux_ref[...] = jnp.dot(h, aw_ref[...], preferred_element_type=jnp.float32)


def _tc_entry1(x, w, aw):
    return pl.pallas_call(
        _entry1_body,
        grid=(5,),
        in_specs=[
            pl.BlockSpec((2000, H), lambda i: (i, 0)),
            pl.BlockSpec((H, H), lambda i: (0, 0)),
            pl.BlockSpec((H, 8), lambda i: (0, 0)),
        ],
        out_specs=[
            pl.BlockSpec((2000, HQ), lambda i: (i, 0)),
            pl.BlockSpec((2000, HQ), lambda i: (i, 0)),
            pl.BlockSpec((2000, HQ), lambda i: (i, 0)),
            pl.BlockSpec((2000, HQ), lambda i: (i, 0)),
            pl.BlockSpec((2000, 8), lambda i: (i, 0)),
        ],
        out_shape=[
            jax.ShapeDtypeStruct((N, HQ), jnp.float32),
            jax.ShapeDtypeStruct((N, HQ), jnp.float32),
            jax.ShapeDtypeStruct((N, HQ), jnp.float32),
            jax.ShapeDtypeStruct((N, HQ), jnp.float32),
            jax.ShapeDtypeStruct((N, 8), jnp.float32),
        ],
    )(x, w, aw)


def _tc_entry2(p, w, aw):
    return pl.pallas_call(
        _entry2_body,
        grid=(5,),
        in_specs=[
            pl.BlockSpec((2000, HQ), lambda i: (i, 0)),
            pl.BlockSpec((2000, HQ), lambda i: (i + 5, 0)),
            pl.BlockSpec((2000, HQ), lambda i: (i + 10, 0)),
            pl.BlockSpec((2000, HQ), lambda i: (i + 15, 0)),
            pl.BlockSpec((HQ, H), lambda i: (0, 0)),
            pl.BlockSpec((HQ, H), lambda i: (1, 0)),
            pl.BlockSpec((HQ, H), lambda i: (2, 0)),
            pl.BlockSpec((HQ, H), lambda i: (3, 0)),
            pl.BlockSpec((H, 8), lambda i: (0, 0)),
        ],
        out_specs=[
            pl.BlockSpec((2000, HQ), lambda i: (i, 0)),
            pl.BlockSpec((2000, HQ), lambda i: (i, 0)),
            pl.BlockSpec((2000, HQ), lambda i: (i, 0)),
            pl.BlockSpec((2000, HQ), lambda i: (i, 0)),
            pl.BlockSpec((2000, 8), lambda i: (i, 0)),
        ],
        out_shape=[
            jax.ShapeDtypeStruct((N, HQ), jnp.float32),
            jax.ShapeDtypeStruct((N, HQ), jnp.float32),
            jax.ShapeDtypeStruct((N, HQ), jnp.float32),
            jax.ShapeDtypeStruct((N, HQ), jnp.float32),
            jax.ShapeDtypeStruct((N, 8), jnp.float32),
        ],
    )(p, p, p, p, w, w, w, w, aw)


def _final_body(p0_ref, p1_ref, p2_ref, p3_ref, b_ref, w1_ref, b1_ref,
                w2_ref, b2_ref, o_ref):
    a = jnp.concatenate([_selu(p0_ref[...]), _selu(p1_ref[...]),
                         _selu(p2_ref[...]), _selu(p3_ref[...])], axis=1)
    bt = b_ref[...]
    gid = lax.broadcasted_iota(jnp.int32, (G, N), 0).astype(jnp.float32)
    P = jnp.where(gid == bt, 1.0, 0.0).astype(jnp.float32)
    cnt = jnp.sum(P, axis=1, keepdims=True)
    pooled = jnp.dot(P, a, preferred_element_type=jnp.float32)
    pooled = pooled / jnp.maximum(cnt, 1.0)
    hm = _selu(jnp.dot(pooled, w1_ref[...],
                       preferred_element_type=jnp.float32) + b1_ref[...])
    logits = jnp.dot(hm, w2_ref[...],
                     preferred_element_type=jnp.float32) + b2_ref[...]
    mx = jnp.max(logits, axis=1, keepdims=True)
    z = logits - mx
    o_ref[...] = z - jnp.log(jnp.sum(jnp.exp(z), axis=1, keepdims=True))


def _tc_final(p, batch2d, w1, b1, w2, b2):
    return pl.pallas_call(
        _final_body,
        grid=(1,),
        in_specs=[
            pl.BlockSpec((N, HQ), lambda i: (0, 0)),
            pl.BlockSpec((N, HQ), lambda i: (1, 0)),
            pl.BlockSpec((N, HQ), lambda i: (2, 0)),
            pl.BlockSpec((N, HQ), lambda i: (3, 0)),
            pl.BlockSpec((1, N), lambda i: (0, 0)),
            pl.BlockSpec((H, H), lambda i: (0, 0)),
            pl.BlockSpec((1, H), lambda i: (0, 0)),
            pl.BlockSpec((H, C), lambda i: (0, 0)),
            pl.BlockSpec((1, C), lambda i: (0, 0)),
        ],
        out_specs=pl.BlockSpec((G, C), lambda i: (0, 0)),
        out_shape=jax.ShapeDtypeStruct((G, C), jnp.float32),
    )(p, p, p, p, batch2d, w1, b1, w2, b2)


_GAT_KERNEL_CACHE = []


def _build_gat_kernel():
    mesh = plsc.VectorSubcoreMesh(core_axis_name="c", subcore_axis_name="s")

    @functools.partial(
        pl.kernel,
        out_type=jax.ShapeDtypeStruct((4 * N, HQ), jnp.float32),
        mesh=mesh,
        compiler_params=pltpu.CompilerParams(
            needs_layout_passes=False, use_tc_tiling_on_sc=False),
        scratch_types=[
            pltpu.VMEM((NR, 16), jnp.float32),
            pltpu.VMEM((NR, 16), jnp.float32),
            pltpu.VMEM((NRP, 16), jnp.float32),
            pltpu.VMEM((ET,), jnp.int32),
            pltpu.VMEM((ET,), jnp.int32),
            pltpu.VMEM((ET // 16, 16), jnp.float32),
            pltpu.VMEM((KC, HQ), jnp.float32),
            pltpu.VMEM((KC, HQ), jnp.float32),
            pltpu.VMEM((KC,), jnp.int32),
            pltpu.VMEM((KC,), jnp.int32),
            pltpu.VMEM((KC,), jnp.int32),
            pltpu.VMEM((KC,), jnp.int32),
            pltpu.VMEM((KT,), jnp.int32),
            pltpu.VMEM((KT,), jnp.int32),
            pltpu.VMEM((NRP,), jnp.int32),
            pltpu.VMEM((16, 16), jnp.float32),
            pltpu.VMEM_SHARED((NRP, 16), jnp.float32),
            pltpu.VMEM_SHARED((N, HQ), jnp.float32),
            pltpu.SemaphoreType.DMA,
            pltpu.SemaphoreType.DMA,
            pltpu.SemaphoreType.DMA,
            pltpu.SemaphoreType.DMA,
        ],
    )
    def k(hq0_hbm, hq1_hbm, hq2_hbm, hq3_hbm, src_hbm, dst_hbm,
          el_hbm, er_hbm, out_hbm,
          el_v, er_v, den_v, src_v, dst_v, ee_v, rows0_v, rows1_v,
          sidx0_v, sidx1_v, didx0_v, didx1_v, sidxt_v, didxt_v,
          irow_v, scr_v, dens_s, outs_s, gsem0, gsem1, ssem0, ssem1):
        c = lax.axis_index("c")
        t = lax.axis_index("s")
        i16 = lax.iota(jnp.int32, 16)
        zf16 = jnp.zeros((16,), jnp.float32)

        pltpu.sync_copy(el_hbm, el_v)
        pltpu.sync_copy(er_hbm, er_v)
        eb = t * ET
        pltpu.sync_copy(src_hbm.at[pl.ds(eb, ET)], src_v)
        pltpu.sync_copy(dst_hbm.at[pl.ds(eb, ET)], dst_v)

        def zden(r, _):
            den_v[r, :] = zf16
            return 0
        lax.fori_loop(0, NRP, zden, 0)


        for g in range(NRP // 16):
            irow_v[pl.ds(16 * g, 16)] = i16 + 16 * g

        @pl.when(t == 0)
        def _():
            pltpu.sync_copy(den_v, dens_s)

        def mrow(r, carry):
            ml, mr = carry
            return (jnp.maximum(ml, el_v[r, :]), jnp.maximum(mr, er_v[r, :]))
        accl, accr = lax.fori_loop(
            0, NR, mrow,
            (jnp.full((16,), -3e38, jnp.float32),
             jnp.full((16,), -3e38, jnp.float32)))
        z16 = jnp.zeros((16,), jnp.int32)
        scr_v[0, :] = accl
        scr_v[1, :] = accr
        ml = jnp.full((16,), -3e38, jnp.float32)
        mr = jnp.full((16,), -3e38, jnp.float32)
        for j in range(16):
            jj = jnp.full((16,), j, jnp.int32)
            ml = jnp.maximum(ml, plsc.load_gather(scr_v, [z16, jj]))
            mr = jnp.maximum(mr, plsc.load_gather(scr_v, [z16 + 1, jj]))
        msum = ml + mr
        m = jnp.maximum(msum, 0.2 * msum) - 30.0

        plsc.subcore_barrier()

        def pa(g, _):
            s16 = src_v[pl.ds(g * 16, 16)]
            d16 = dst_v[pl.ds(g * 16, 16)]
            srow = lax.shift_right_logical(s16, 4)
            scol = lax.bitwise_and(s16, 15)
            drow = lax.shift_right_logical(d16, 4)
            dcol = lax.bitwise_and(d16, 15)
            elv = plsc.load_gather(el_v, [srow, scol])
            erv = plsc.load_gather(er_v, [drow, dcol])
            xv = elv + erv
            ev = jnp.where(xv > 0, xv, 0.2 * xv)
            ee = jnp.exp(ev - m)
            ee_v[g, :] = ee
            plsc.addupdate_scatter(den_v, [drow, dcol], ee)
            return 0
        lax.fori_loop(0, ET // 16, pa, 0)

        pltpu.sync_copy(den_v, dens_s.at[irow_v], add=True)
        plsc.subcore_barrier()
        pltpu.sync_copy(dens_s, den_v)

        def pb(g, _):
            ee = ee_v[g, :]
            d16 = dst_v[pl.ds(g * 16, 16)]
            drow = lax.shift_right_logical(d16, 4)
            dcol = lax.bitwise_and(d16, 15)
            dv = plsc.load_gather(den_v, [drow, dcol])
            ee_v[g, :] = ee / (dv + 1e-9)
            return 0
        lax.fori_loop(0, ET // 16, pb, 0)

        ob = t * RB
        NPAIR = ET // (2 * KC)

        def zrows(r, _):
            for j in range(HQ // 16):
                rows0_v[r, pl.ds(16 * j, 16)] = zf16
            return 0

        def fill_idx(sbuf, dbuf, off):
            for u in range(KC // 16):
                sbuf[pl.ds(16 * u, 16)] = src_v[pl.ds(off + 16 * u, 16)]
                dbuf[pl.ds(16 * u, 16)] = dst_v[pl.ds(off + 16 * u, 16)]

        def scale_rows(rbuf, off):
            def scale(i4, _):
                for s in range(4):
                    i = i4 * 4 + s
                    ea = off + i
                    av = plsc.load_gather(
                        ee_v, [jnp.full((16,),
                                        lax.shift_right_logical(ea, 4),
                                        jnp.int32),
                               jnp.full((16,), lax.bitwise_and(ea, 15),
                                        jnp.int32)])
                    for j in range(HQ // 16):
                        blk = rbuf[i, pl.ds(16 * j, 16)]
                        rbuf[i, pl.ds(16 * j, 16)] = blk * av
                return 0
            lax.fori_loop(0, KC // 4, scale, 0)

        for half in range(2):
            lax.fori_loop(0, KC, zrows, 0)
            nzf = 0
            for q in range(RB // KC):
                pltpu.async_copy(rows0_v, outs_s.at[pl.ds(ob + q * KC, KC)],
                                 gsem0)
                nzf += 1
            rem = RB - (RB // KC) * KC
            if rem:
                pltpu.async_copy(rows0_v.at[pl.ds(0, rem)],
                                 outs_s.at[pl.ds(ob + (RB // KC) * KC, rem)],
                                 gsem1)

            @pl.when(t == NTILES - 1)
            def _():
                pltpu.async_copy(
                    rows0_v.at[pl.ds(0, N - NTILES * RB)],
                    outs_s.at[pl.ds(NTILES * RB, N - NTILES * RB)], ssem0)

            for q in range(nzf):
                pltpu.make_async_copy(
                    rows0_v, outs_s.at[pl.ds(ob, KC)], gsem0).wait()
            if rem:
                pltpu.make_async_copy(
                    rows0_v.at[pl.ds(0, rem)],
                    outs_s.at[pl.ds(ob, rem)], gsem1).wait()

            @pl.when(t == NTILES - 1)
            def _():
                pltpu.make_async_copy(
                    rows0_v.at[pl.ds(0, N - NTILES * RB)],
                    outs_s.at[pl.ds(NTILES * RB, N - NTILES * RB)],
                    ssem0).wait()

            plsc.subcore_barrier()

            def start_gather(sbuf, rbuf, gsem):
                if half == 0:
                    @pl.when(c == 0)
                    def _():
                        pltpu.async_copy(hq0_hbm.at[sbuf], rbuf, gsem)

                    @pl.when(c == 1)
                    def _():
                        pltpu.async_copy(hq2_hbm.at[sbuf], rbuf, gsem)
                else:
                    @pl.when(c == 0)
                    def _():
                        pltpu.async_copy(hq1_hbm.at[sbuf], rbuf, gsem)

                    @pl.when(c == 1)
                    def _():
                        pltpu.async_copy(hq3_hbm.at[sbuf], rbuf, gsem)

            fill_idx(sidx0_v, didx0_v, 0)
            start_gather(sidx0_v, rows0_v, gsem0)

            def pc2(kk2, _):
                base0 = kk2 * (2 * KC)
                base1 = base0 + KC

                @pl.when(kk2 > 0)
                def _():
                    pltpu.make_async_copy(
                        hq0_hbm.at[sidx1_v], rows1_v, ssem1).wait()
                fill_idx(sidx1_v, didx1_v, base1)
                start_gather(sidx1_v, rows1_v, gsem1)

                pltpu.make_async_copy(
                    hq0_hbm.at[sidx0_v], rows0_v, gsem0).wait()
                scale_rows(rows0_v, base0)
                pltpu.async_copy(rows0_v, outs_s.at[didx0_v], ssem0,
                                 add=True)

                @pl.when(kk2 < NPAIR - 1)
                def _():
                    pltpu.make_async_copy(
                        hq0_hbm.at[sidx0_v], rows0_v, ssem0).wait()
                    fill_idx(sidx0_v, didx0_v, base0 + 2 * KC)
                    start_gather(sidx0_v, rows0_v, gsem0)

                pltpu.make_async_copy(
                    hq0_hbm.at[sidx1_v], rows1_v, gsem1).wait()
                scale_rows(rows1_v, base1)
                pltpu.async_copy(rows1_v, outs_s.at[didx1_v], ssem1,
                                 add=True)
                return 0
            lax.fori_loop(0, NPAIR, pc2, 0)

            pltpu.make_async_copy(hq0_hbm.at[sidx0_v], rows0_v, ssem0).wait()
            pltpu.make_async_copy(hq0_hbm.at[sidx1_v], rows1_v, ssem1).wait()

            tb = NPAIR * 2 * KC
            for u in range(KT // 16):
                sidxt_v[pl.ds(16 * u, 16)] = src_v[pl.ds(tb + 16 * u, 16)]
                didxt_v[pl.ds(16 * u, 16)] = dst_v[pl.ds(tb + 16 * u, 16)]
            if half == 0:
                @pl.when(c == 0)
                def _():
                    pltpu.async_copy(hq0_hbm.at[sidxt_v],
                                     rows0_v.at[pl.ds(0, KT)], gsem0).wait()

                @pl.when(c == 1)
                def _():
                    pltpu.async_copy(hq2_hbm.at[sidxt_v],
                                     rows0_v.at[pl.ds(0, KT)], gsem0).wait()
            else:
                @pl.when(c == 0)
                def _():
                    pltpu.async_copy(hq1_hbm.at[sidxt_v],
                                     rows0_v.at[pl.ds(0, KT)], gsem0).wait()

                @pl.when(c == 1)
                def _():
                    pltpu.async_copy(hq3_hbm.at[sidxt_v],
                                     rows0_v.at[pl.ds(0, KT)], gsem0).wait()

            def scalet(i4, _):
                for s in range(4):
                    i = i4 * 4 + s
                    ea = tb + i
                    av = plsc.load_gather(
                        ee_v, [jnp.full((16,),
                                        lax.shift_right_logical(ea, 4),
                                        jnp.int32),
                               jnp.full((16,), lax.bitwise_and(ea, 15),
                                        jnp.int32)])
                    for j in range(HQ // 16):
                        blk = rows0_v[i, pl.ds(16 * j, 16)]
                        rows0_v[i, pl.ds(16 * j, 16)] = blk * av
                return 0
            lax.fori_loop(0, KT // 4, scalet, 0)
            pltpu.sync_copy(rows0_v.at[pl.ds(0, KT)], outs_s.at[didxt_v],
                            add=True)
            plsc.subcore_barrier()

            qb = (2 * c + half) * N
            pltpu.sync_copy(outs_s.at[pl.ds(ob, RB)],
                            out_hbm.at[pl.ds(qb + ob, RB)])

            @pl.when(t == NTILES - 1)
            def _():
                pltpu.sync_copy(
                    outs_s.at[pl.ds(NTILES * RB, N - NTILES * RB)],
                    out_hbm.at[pl.ds(qb + NTILES * RB, N - NTILES * RB)])

            plsc.subcore_barrier()

    return k


def _gat_sc(hq0, hq1, hq2, hq3, src, dst, el2d, er2d):
    if not _GAT_KERNEL_CACHE:
        _GAT_KERNEL_CACHE.append(_build_gat_kernel())
    return _GAT_KERNEL_CACHE[0](hq0, hq1, hq2, hq3, src, dst, el2d, er2d)


def kernel(x, edge_index, batch, Wg1, al1, ar1, Wg2, al2, ar2, W1, b1, W2, b2):
    src = edge_index[0]
    dst = edge_index[1]
    zcol = jnp.zeros((H,), jnp.float32)
    aw1 = jnp.stack([al1, ar1, zcol, zcol, zcol, zcol, zcol, zcol], axis=1)
    aw2 = jnp.stack([al2, ar2, zcol, zcol, zcol, zcol, zcol, zcol], axis=1)

    q10, q11, q12, q13, aux1 = _tc_entry1(x, Wg1, aw1)
    el1 = aux1[:, 0].reshape(NR, 16)
    er1 = aux1[:, 1].reshape(NR, 16)
    p1 = _gat_sc(q10, q11, q12, q13, src, dst, el1, er1)

    q20, q21, q22, q23, aux2 = _tc_entry2(p1, Wg2, aw2)
    el2 = aux2[:, 0].reshape(NR, 16)
    er2 = aux2[:, 1].reshape(NR, 16)
    p2 = _gat_sc(q20, q21, q22, q23, src, dst, el2, er2)

    batchf = batch.astype(jnp.float32).reshape(1, N)
    return _tc_final(p2, batchf, W1,
                     b1.reshape(1, H), W2, b2.reshape(1, C))
